# baseline (speedup 1.0000x reference)
import jax
import jax.numpy as jnp
from jax import lax
from jax.experimental import pallas as pl
from jax.experimental.pallas import tpu as pltpu
from jax.experimental.pallas import tpu_sc as plsc

HIDDEN = 2048
NUM_EXPERTS = 64
TOP_K = 2
ROUTED_SCALING = 1.0

TSUB = 512
NBUF = 8

N_TOKENS = 16384
N_WORKERS = 32
TOK_PER_W = N_TOKENS // N_WORKERS
LANES = 16


def _gate_body(x_hbm, w_ref, logits_ref, lt_ref, xbuf, sems):
    i = pl.program_id(0)
    nblk = pl.num_programs(0)
    ib = lax.rem(i, NBUF)
    ip = lax.rem(i + NBUF - 1, NBUF)

    @pl.when(i == 0)
    def _prime():
        for b in range(NBUF - 1):
            pltpu.make_async_copy(
                x_hbm.at[pl.ds(b * TSUB, TSUB), :],
                xbuf.at[b],
                sems.at[b],
            ).start()

    @pl.when(i + NBUF - 1 < nblk)
    def _prefetch():
        pltpu.make_async_copy(
            x_hbm.at[pl.ds((i + NBUF - 1) * TSUB, TSUB), :],
            xbuf.at[ip],
            sems.at[ip],
        ).start()

    pltpu.make_async_copy(
        x_hbm.at[pl.ds(i * TSUB, TSUB), :],
        xbuf.at[ib],
        sems.at[ib],
    ).wait()

    x = xbuf[ib]
    w = w_ref[...]
    logits = jax.lax.dot_general(
        x, w, (((1,), (0,)), ((), ())),
        precision=jax.lax.Precision.DEFAULT,
        preferred_element_type=jnp.float32,
    )
    logits_ref[...] = logits
    lt_ref[...] = logits.T


def _route_body(lt_hbm, i1_hbm, i2_hbm, w1_hbm, w2_hbm,
                buf, i1b, i2b, w1b, w2b):
    wid = lax.axis_index("s") * 2 + lax.axis_index("c")
    base = wid * TOK_PER_W
    pltpu.sync_copy(lt_hbm.at[:, pl.ds(base, TOK_PER_W)], buf)

    @pl.loop(0, TOK_PER_W // LANES)
    def _group(g):
        off = g * LANES
        neg_inf = jnp.full((LANES,), -jnp.inf, jnp.float32)
        zero_i = jnp.zeros((LANES,), jnp.int32)
        v1, i1 = neg_inf, zero_i
        v2, i2 = neg_inf, zero_i
        for e in range(NUM_EXPERTS):
            x = buf[e, pl.ds(off, LANES)]
            e_vec = jnp.full((LANES,), e, jnp.int32)
            gt1 = x > v1
            gt2 = x > v2
            i2 = jnp.where(gt1, i1, jnp.where(gt2, e_vec, i2))
            v2 = jnp.where(gt1, v1, jnp.where(gt2, x, v2))
            i1 = jnp.where(gt1, e_vec, i1)
            v1 = jnp.where(gt1, x, v1)
        e2 = jnp.exp(v2 - v1)
        w1 = ROUTED_SCALING / (1.0 + e2)
        i1b[pl.ds(off, LANES)] = i1
        i2b[pl.ds(off, LANES)] = i2
        w1b[pl.ds(off, LANES)] = w1
        w2b[pl.ds(off, LANES)] = e2 * w1

    pltpu.sync_copy(i1b, i1_hbm.at[pl.ds(base, TOK_PER_W)])
    pltpu.sync_copy(i2b, i2_hbm.at[pl.ds(base, TOK_PER_W)])
    pltpu.sync_copy(w1b, w1_hbm.at[pl.ds(base, TOK_PER_W)])
    pltpu.sync_copy(w2b, w2_hbm.at[pl.ds(base, TOK_PER_W)])


def kernel(hidden_states, gate_weight):
    b, s, h = hidden_states.shape
    n = b * s
    x = hidden_states.reshape(n, h)
    wt = gate_weight.T

    grid = (n // TSUB,)
    logits, lt = pl.pallas_call(
        _gate_body,
        grid=grid,
        in_specs=[
            pl.BlockSpec(memory_space=pltpu.MemorySpace.HBM),
            pl.BlockSpec((h, NUM_EXPERTS), lambda i: (0, 0)),
        ],
        out_specs=[
            pl.BlockSpec((TSUB, NUM_EXPERTS), lambda i: (i, 0)),
            pl.BlockSpec((NUM_EXPERTS, TSUB), lambda i: (0, i)),
        ],
        out_shape=[
            jax.ShapeDtypeStruct((n, NUM_EXPERTS), jnp.float32),
            jax.ShapeDtypeStruct((NUM_EXPERTS, n), jnp.float32),
        ],
        scratch_shapes=[
            pltpu.VMEM((NBUF, TSUB, HIDDEN), jnp.float32),
            pltpu.SemaphoreType.DMA((NBUF,)),
        ],
        compiler_params=pltpu.CompilerParams(
            dimension_semantics=("arbitrary",),
        ),
    )(x, wt)

    mesh = plsc.VectorSubcoreMesh(core_axis_name="c", subcore_axis_name="s")
    route = pl.kernel(
        _route_body,
        mesh=mesh,
        out_type=[
            jax.ShapeDtypeStruct((n,), jnp.int32),
            jax.ShapeDtypeStruct((n,), jnp.int32),
            jax.ShapeDtypeStruct((n,), jnp.float32),
            jax.ShapeDtypeStruct((n,), jnp.float32),
        ],
        scratch_types=[
            pltpu.VMEM((NUM_EXPERTS, TOK_PER_W), jnp.float32),
            pltpu.VMEM((TOK_PER_W,), jnp.int32),
            pltpu.VMEM((TOK_PER_W,), jnp.int32),
            pltpu.VMEM((TOK_PER_W,), jnp.float32),
            pltpu.VMEM((TOK_PER_W,), jnp.float32),
        ],
    )
    i1, i2, w1, w2 = route(lt)
    idx = jnp.concatenate([i1[:, None], i2[:, None]], axis=1)
    tw = jnp.concatenate([w1[:, None], w2[:, None]], axis=1)
    return (idx, tw, logits)


# speedup vs baseline: 1.1024x; 1.1024x over previous
import jax
import jax.numpy as jnp
from jax import lax
from jax.experimental import pallas as pl
from jax.experimental.pallas import tpu as pltpu

HIDDEN = 2048
NUM_EXPERTS = 64
TOP_K = 2
ROUTED_SCALING = 1.0

TSUB = 512
NBUF = 8


def _router_body(x_hbm, w_ref, logits_ref, idx_ref, tw_ref, xbuf, sems):
    i = pl.program_id(0)
    nblk = pl.num_programs(0)
    ib = lax.rem(i, NBUF)
    ip = lax.rem(i + NBUF - 1, NBUF)

    @pl.when(i == 0)
    def _prime():
        for b in range(NBUF - 1):
            pltpu.make_async_copy(
                x_hbm.at[pl.ds(b * TSUB, TSUB), :],
                xbuf.at[b],
                sems.at[b],
            ).start()

    @pl.when(i + NBUF - 1 < nblk)
    def _prefetch():
        pltpu.make_async_copy(
            x_hbm.at[pl.ds((i + NBUF - 1) * TSUB, TSUB), :],
            xbuf.at[ip],
            sems.at[ip],
        ).start()

    pltpu.make_async_copy(
        x_hbm.at[pl.ds(i * TSUB, TSUB), :],
        xbuf.at[ib],
        sems.at[ib],
    ).wait()

    x = xbuf[ib]
    w = w_ref[...]
    logits = jax.lax.dot_general(
        x, w, (((1,), (0,)), ((), ())),
        precision=jax.lax.Precision.DEFAULT,
        preferred_element_type=jnp.float32,
    )
    logits_ref[...] = logits

    iota = lax.broadcasted_iota(jnp.int32, logits.shape, 1)
    v1 = jnp.max(logits, axis=1, keepdims=True)
    i1 = jnp.min(jnp.where(logits == v1, iota, NUM_EXPERTS), axis=1,
                 keepdims=True)
    masked = jnp.where(iota == i1, -jnp.inf, logits)
    v2 = jnp.max(masked, axis=1, keepdims=True)
    i2 = jnp.min(jnp.where(masked == v2, iota, NUM_EXPERTS), axis=1,
                 keepdims=True)

    e2 = jnp.exp(v2 - v1)
    scale = ROUTED_SCALING / (1.0 + e2)
    tw_ref[...] = jnp.concatenate([scale, e2 * scale], axis=1)
    idx_ref[...] = jnp.concatenate([i1, i2], axis=1)


def kernel(hidden_states, gate_weight):
    b, s, h = hidden_states.shape
    n = b * s
    x = hidden_states.reshape(n, h)
    wt = gate_weight.T

    grid = (n // TSUB,)
    logits, idx, tw = pl.pallas_call(
        _router_body,
        grid=grid,
        in_specs=[
            pl.BlockSpec(memory_space=pltpu.MemorySpace.HBM),
            pl.BlockSpec((h, NUM_EXPERTS), lambda i: (0, 0)),
        ],
        out_specs=[
            pl.BlockSpec((TSUB, NUM_EXPERTS), lambda i: (i, 0)),
            pl.BlockSpec((TSUB, TOP_K), lambda i: (i, 0)),
            pl.BlockSpec((TSUB, TOP_K), lambda i: (i, 0)),
        ],
        out_shape=[
            jax.ShapeDtypeStruct((n, NUM_EXPERTS), jnp.float32),
            jax.ShapeDtypeStruct((n, TOP_K), jnp.int32),
            jax.ShapeDtypeStruct((n, TOP_K), jnp.float32),
        ],
        scratch_shapes=[
            pltpu.VMEM((NBUF, TSUB, HIDDEN), jnp.float32),
            pltpu.SemaphoreType.DMA((NBUF,)),
        ],
        compiler_params=pltpu.CompilerParams(
            dimension_semantics=("arbitrary",),
        ),
    )(x, wt)
    return (idx, tw, logits)
